# baseline (speedup 1.0000x reference)
import jax
import jax.numpy as jnp
from jax.experimental import pallas as pl
from jax.experimental.pallas import tpu as pltpu

HIDDEN = 4096
EXPERTS = 64
TOPK = 8
SOFTCAP = 30.0
CHUNK = 512
NBUF = 4


def _copy(x_hbm, buf, sems, c, slot):
    return pltpu.make_async_copy(
        x_hbm.at[pl.ds(c * CHUNK, CHUNK)], buf.at[slot], sems.at[slot]
    )


def _gate_kernel(n_chunks, w_ref, x_hbm, wout_ref, iout_ref, buf, sems):
    w = w_ref[...]
    iota = jax.lax.broadcasted_iota(jnp.int32, (EXPERTS, CHUNK), 0).astype(jnp.float32)

    for b in range(NBUF):
        _copy(x_hbm, buf, sems, b, b).start()

    def outer(o, carry):
        for b in range(NBUF):
            c = o * NBUF + b
            _copy(x_hbm, buf, sems, c, b).wait()
            logits = jax.lax.dot_general(
                w, buf[b], (((1,), (1,)), ((), ())),
                preferred_element_type=jnp.float32,
            )
            logits = jnp.tanh(logits * (1.0 / SOFTCAP)) * SOFTCAP
            cur = logits
            vals = []
            idxs = []
            for _ in range(TOPK):
                m = jnp.max(cur, axis=0, keepdims=True)
                sel = jnp.min(
                    jnp.where(cur == m, iota, float(EXPERTS)), axis=0, keepdims=True
                )
                vals.append(m)
                idxs.append(sel)
                cur = jnp.where(iota == sel, -jnp.inf, cur)
            v = jnp.concatenate(vals, axis=0)
            s = jnp.concatenate(idxs, axis=0)
            e = jnp.exp(v - v[0:1])
            off = pl.multiple_of(c * CHUNK, CHUNK)
            wout_ref[:, pl.ds(off, CHUNK)] = e / jnp.sum(e, axis=0, keepdims=True)
            iout_ref[:, pl.ds(off, CHUNK)] = s.astype(jnp.int32)
            nc = c + NBUF

            @pl.when(nc < n_chunks)
            def _():
                _copy(x_hbm, buf, sems, nc, b).start()

        return carry

    jax.lax.fori_loop(0, n_chunks // NBUF, outer, 0)


def kernel(hidden_states, gate_w):
    b, seq, h = hidden_states.shape
    n_tok = b * seq
    n_chunks = n_tok // CHUNK
    x = hidden_states.reshape(n_tok, h)
    import functools

    wout, iout = pl.pallas_call(
        functools.partial(_gate_kernel, n_chunks),
        in_specs=[
            pl.BlockSpec((EXPERTS, h), lambda: (0, 0)),
            pl.BlockSpec(memory_space=pl.ANY),
        ],
        out_specs=[
            pl.BlockSpec((TOPK, n_tok), lambda: (0, 0)),
            pl.BlockSpec((TOPK, n_tok), lambda: (0, 0)),
        ],
        out_shape=[
            jax.ShapeDtypeStruct((TOPK, n_tok), jnp.float32),
            jax.ShapeDtypeStruct((TOPK, n_tok), jnp.int32),
        ],
        scratch_shapes=[
            pltpu.VMEM((NBUF, CHUNK, HIDDEN), jnp.float32),
            pltpu.SemaphoreType.DMA((NBUF,)),
        ],
    )(gate_w, x)
    return wout.T, iout.T


# speedup vs baseline: 1.0396x; 1.0396x over previous
import jax
import jax.numpy as jnp
from jax.experimental import pallas as pl
from jax.experimental.pallas import tpu as pltpu

HIDDEN = 4096
EXPERTS = 64
TOPK = 8
SOFTCAP = 30.0
BLOCK_T = 1024


def _gate_kernel(w_ref, x_ref, wout_ref, iout_ref):
    w = w_ref[...]
    x = x_ref[...]
    logits = jax.lax.dot_general(
        w, x, (((1,), (1,)), ((), ())), preferred_element_type=jnp.float32
    )
    logits = jnp.tanh(logits * (1.0 / SOFTCAP)) * SOFTCAP

    t = logits.shape[1]
    iota = jax.lax.broadcasted_iota(jnp.int32, (EXPERTS, t), 0).astype(jnp.float32)
    cur = logits
    vals = []
    idxs = []
    for _ in range(TOPK):
        m = jnp.max(cur, axis=0, keepdims=True)
        sel = jnp.min(jnp.where(cur == m, iota, float(EXPERTS)), axis=0, keepdims=True)
        vals.append(m)
        idxs.append(sel)
        cur = jnp.where(iota == sel, -jnp.inf, cur)
    v = jnp.concatenate(vals, axis=0)
    s = jnp.concatenate(idxs, axis=0)
    e = jnp.exp(v - v[0:1])
    wout_ref[...] = e / jnp.sum(e, axis=0, keepdims=True)
    iout_ref[...] = s.astype(jnp.int32)


def kernel(hidden_states, gate_w):
    b, seq, h = hidden_states.shape
    n_tok = b * seq
    x = hidden_states.reshape(n_tok, h)
    grid = (n_tok // BLOCK_T,)
    wout, iout = pl.pallas_call(
        _gate_kernel,
        grid=grid,
        in_specs=[
            pl.BlockSpec((EXPERTS, h), lambda i: (0, 0)),
            pl.BlockSpec((BLOCK_T, h), lambda i: (i, 0)),
        ],
        out_specs=[
            pl.BlockSpec((TOPK, BLOCK_T), lambda i: (0, i)),
            pl.BlockSpec((TOPK, BLOCK_T), lambda i: (0, i)),
        ],
        out_shape=[
            jax.ShapeDtypeStruct((TOPK, n_tok), jnp.float32),
            jax.ShapeDtypeStruct((TOPK, n_tok), jnp.int32),
        ],
        compiler_params=pltpu.CompilerParams(
            dimension_semantics=("parallel",),
        ),
    )(gate_w, x)
    return wout.T, iout.T
